# Initial kernel scaffold; baseline (speedup 1.0000x reference)
import jax
import jax.numpy as jnp
from jax.experimental import pallas as pl


def kernel(x, gating_output, gate_up_proj, down_proj):
    raise NotImplementedError("write your pallas kernel here")


# speedup vs baseline: 2.2757x; 2.2757x over previous
import functools

import jax
import jax.numpy as jnp
from jax import lax
from jax.experimental import pallas as pl
from jax.experimental.pallas import tpu as pltpu

E = 8
TOPK = 2
D = 1024
FF = 1024
T = 256

FB = 512
NF = FF // FB


def _combine_from_logits(g):
    iota = lax.broadcasted_iota(jnp.int32, g.shape, 1)
    m1 = jnp.max(g, axis=1, keepdims=True)
    i1 = jnp.min(jnp.where(g == m1, iota, E), axis=1, keepdims=True)
    mask1 = iota == i1
    g_rest = jnp.where(mask1, -jnp.inf, g)
    m2 = jnp.max(g_rest, axis=1, keepdims=True)
    i2 = jnp.min(jnp.where(g_rest == m2, iota, E), axis=1, keepdims=True)
    mask2 = iota == i2
    w1 = jax.nn.sigmoid(m1 - m2)
    w2 = 1.0 - w1
    return jnp.where(mask1, w1, 0.0) + jnp.where(mask2, w2, 0.0)


def _moe_body(x_ref, gating_ref, gate_ref, up_ref, down_ref, out_ref,
              acc_ref, combine_ref):
    e = pl.program_id(0)
    f = pl.program_id(1)

    @pl.when(jnp.logical_and(e == 0, f == 0))
    def _():
        combine_ref[...] = _combine_from_logits(gating_ref[...])

    x = x_ref[...]
    gate_w = gate_ref[0]
    up_w = up_ref[0]
    down_w = down_ref[0]

    nt = (((1,), (1,)), ((), ()))
    gg = lax.dot_general(x, gate_w, nt, preferred_element_type=jnp.float32)
    uu = lax.dot_general(x, up_w, nt, preferred_element_type=jnp.float32)
    h = gg * jax.nn.sigmoid(gg) * uu
    yb = lax.dot_general(h, down_w, nt, preferred_element_type=jnp.float32)

    @pl.when(f == 0)
    def _():
        acc_ref[...] = yb

    @pl.when(f != 0)
    def _():
        acc_ref[...] += yb

    @pl.when(f == NF - 1)
    def _():
        cm = combine_ref[...]
        sel = lax.broadcasted_iota(jnp.int32, cm.shape, 1) == e
        col = jnp.sum(jnp.where(sel, cm, 0.0), axis=1, keepdims=True)
        contrib = acc_ref[...] * col

        @pl.when(e == 0)
        def _():
            out_ref[...] = contrib

        @pl.when(e != 0)
        def _():
            out_ref[...] += contrib


@jax.jit
def kernel(x, gating_output, gate_up_proj, down_proj):
    grid = (E, NF)
    out = pl.pallas_call(
        _moe_body,
        grid=grid,
        in_specs=[
            pl.BlockSpec((T, D), lambda e, f: (0, 0)),
            pl.BlockSpec((T, E), lambda e, f: (0, 0)),
            pl.BlockSpec((1, FB, D), lambda e, f: (e, f, 0)),
            pl.BlockSpec((1, FB, D), lambda e, f: (e, NF + f, 0)),
            pl.BlockSpec((1, D, FB), lambda e, f: (e, 0, f)),
        ],
        out_specs=pl.BlockSpec((T, D), lambda e, f: (0, 0)),
        out_shape=jax.ShapeDtypeStruct((T, D), jnp.float32),
        scratch_shapes=[
            pltpu.VMEM((T, D), jnp.float32),
            pltpu.VMEM((T, E), jnp.float32),
        ],
    )(x, gating_output, gate_up_proj, gate_up_proj, down_proj)
    return out


# speedup vs baseline: 2.2785x; 1.0012x over previous
import functools

import jax
import jax.numpy as jnp
from jax import lax
from jax.experimental import pallas as pl
from jax.experimental.pallas import tpu as pltpu

E = 8
TOPK = 2
D = 1024
FF = 1024
T = 256

FB = 512
NF = FF // FB


def _combine_from_logits(g):
    iota = lax.broadcasted_iota(jnp.int32, g.shape, 1)
    m1 = jnp.max(g, axis=1, keepdims=True)
    i1 = jnp.min(jnp.where(g == m1, iota, E), axis=1, keepdims=True)
    mask1 = iota == i1
    g_rest = jnp.where(mask1, -jnp.inf, g)
    m2 = jnp.max(g_rest, axis=1, keepdims=True)
    i2 = jnp.min(jnp.where(g_rest == m2, iota, E), axis=1, keepdims=True)
    mask2 = iota == i2
    w1 = jax.nn.sigmoid(m1 - m2)
    w2 = 1.0 - w1
    return jnp.where(mask1, w1, 0.0) + jnp.where(mask2, w2, 0.0)


def _moe_body(x_ref, gating_ref, gate_ref, up_ref, down_ref, out_ref,
              acc_ref, combine_ref):
    e = pl.program_id(0)
    f = pl.program_id(1)

    @pl.when(jnp.logical_and(e == 0, f == 0))
    def _():
        combine_ref[...] = _combine_from_logits(gating_ref[...])

    x = x_ref[...]
    gate_w = gate_ref[0]
    up_w = up_ref[0]
    down_w = down_ref[0]

    nt = (((1,), (1,)), ((), ()))
    xb = x.astype(jnp.bfloat16)
    gg = lax.dot_general(xb, gate_w.astype(jnp.bfloat16), nt,
                         preferred_element_type=jnp.float32)
    uu = lax.dot_general(xb, up_w.astype(jnp.bfloat16), nt,
                         preferred_element_type=jnp.float32)
    h = gg * jax.nn.sigmoid(gg) * uu
    yb = lax.dot_general(h.astype(jnp.bfloat16), down_w.astype(jnp.bfloat16),
                         nt, preferred_element_type=jnp.float32)

    @pl.when(f == 0)
    def _():
        acc_ref[...] = yb

    @pl.when(f != 0)
    def _():
        acc_ref[...] += yb

    @pl.when(f == NF - 1)
    def _():
        cm = combine_ref[...]
        sel = lax.broadcasted_iota(jnp.int32, cm.shape, 1) == e
        col = jnp.sum(jnp.where(sel, cm, 0.0), axis=1, keepdims=True)
        contrib = acc_ref[...] * col

        @pl.when(e == 0)
        def _():
            out_ref[...] = contrib

        @pl.when(e != 0)
        def _():
            out_ref[...] += contrib


@jax.jit
def kernel(x, gating_output, gate_up_proj, down_proj):
    grid = (E, NF)
    out = pl.pallas_call(
        _moe_body,
        grid=grid,
        in_specs=[
            pl.BlockSpec((T, D), lambda e, f: (0, 0)),
            pl.BlockSpec((T, E), lambda e, f: (0, 0)),
            pl.BlockSpec((1, FB, D), lambda e, f: (e, f, 0)),
            pl.BlockSpec((1, FB, D), lambda e, f: (e, NF + f, 0)),
            pl.BlockSpec((1, D, FB), lambda e, f: (e, 0, f)),
        ],
        out_specs=pl.BlockSpec((T, D), lambda e, f: (0, 0)),
        out_shape=jax.ShapeDtypeStruct((T, D), jnp.float32),
        scratch_shapes=[
            pltpu.VMEM((T, D), jnp.float32),
            pltpu.VMEM((T, E), jnp.float32),
        ],
    )(x, gating_output, gate_up_proj, gate_up_proj, down_proj)
    return out


# speedup vs baseline: 2.4951x; 1.0951x over previous
import jax
import jax.numpy as jnp
from jax import lax
from jax.experimental import pallas as pl
from jax.experimental.pallas import tpu as pltpu

E = 8
TOPK = 2
D = 1024
FF = 1024
T = 256


def _combine_from_logits(g):
    iota = lax.broadcasted_iota(jnp.int32, g.shape, 1)
    m1 = jnp.max(g, axis=1, keepdims=True)
    i1 = jnp.min(jnp.where(g == m1, iota, E), axis=1, keepdims=True)
    mask1 = iota == i1
    g_rest = jnp.where(mask1, -jnp.inf, g)
    m2 = jnp.max(g_rest, axis=1, keepdims=True)
    i2 = jnp.min(jnp.where(g_rest == m2, iota, E), axis=1, keepdims=True)
    mask2 = iota == i2
    w1 = jax.nn.sigmoid(m1 - m2)
    w2 = 1.0 - w1
    return jnp.where(mask1, w1, 0.0) + jnp.where(mask2, w2, 0.0)


def _moe_body(x_ref, gating_ref, gu_ref, down_ref, out_ref, combine_ref):
    e = pl.program_id(0)

    @pl.when(e == 0)
    def _():
        combine_ref[...] = _combine_from_logits(gating_ref[...])

    xb = x_ref[...].astype(jnp.bfloat16)
    gate_w = gu_ref[0, :FF].astype(jnp.bfloat16)
    up_w = gu_ref[0, FF:].astype(jnp.bfloat16)
    down_w = down_ref[0].astype(jnp.bfloat16)

    nt = (((1,), (1,)), ((), ()))
    gg = lax.dot_general(xb, gate_w, nt, preferred_element_type=jnp.float32)
    uu = lax.dot_general(xb, up_w, nt, preferred_element_type=jnp.float32)
    h = gg * jax.nn.sigmoid(gg) * uu
    yb = lax.dot_general(h.astype(jnp.bfloat16), down_w, nt,
                         preferred_element_type=jnp.float32)

    cm = combine_ref[...]
    sel = lax.broadcasted_iota(jnp.int32, cm.shape, 1) == e
    col = jnp.sum(jnp.where(sel, cm, 0.0), axis=1, keepdims=True)
    contrib = yb * col

    @pl.when(e == 0)
    def _():
        out_ref[...] = contrib

    @pl.when(e != 0)
    def _():
        out_ref[...] += contrib


@jax.jit
def kernel(x, gating_output, gate_up_proj, down_proj):
    out = pl.pallas_call(
        _moe_body,
        grid=(E,),
        in_specs=[
            pl.BlockSpec((T, D), lambda e: (0, 0)),
            pl.BlockSpec((T, E), lambda e: (0, 0)),
            pl.BlockSpec((1, 2 * FF, D), lambda e: (e, 0, 0)),
            pl.BlockSpec((1, D, FF), lambda e: (e, 0, 0)),
        ],
        out_specs=pl.BlockSpec((T, D), lambda e: (0, 0)),
        out_shape=jax.ShapeDtypeStruct((T, D), jnp.float32),
        scratch_shapes=[
            pltpu.VMEM((T, E), jnp.float32),
        ],
    )(x, gating_output, gate_up_proj, down_proj)
    return out
